# baseline (speedup 1.0000x reference)
import math

import jax
import jax.numpy as jnp
from jax import lax
from jax.experimental import pallas as pl
from jax.experimental.pallas import tpu as pltpu
from jax.experimental.pallas import tpu_sc as plsc

MODEL_DIM = 64
SCALE = math.sqrt(MODEL_DIM)

NC = 2
NS = 16
NW = NC * NS
LANES = 16

VOCAB = 1000000
RCW = 1024
PADV = ((VOCAB + RCW - 1) // RCW) * RCW

RG = 128
G = 2
UR = RG * G
SEQ = 200
BATCH = 4096
UNITS_PER_SEQ = BATCH // UR
N_UNITS = SEQ * UNITS_PER_SEQ
UPW = N_UNITS // NW


def _emb_body(x_hbm, table_hbm, out_hbm, idx_v, rows_v, phys_v,
              g0, g1, s0, s1, i0, i1):
  wid = lax.axis_index("s") * NC + lax.axis_index("c")
  u0 = wid * UPW

  gsem = (g0, g1)
  ssem = (s0, s1)
  isem = (i0, i1)

  def unit_su(u):
    uu = u0 + u
    return uu // UNITS_PER_SEQ, uu % UNITS_PER_SEQ

  def sync_idx(u, b):
    s, j = unit_su(u)
    pltpu.sync_copy(x_hbm.at[s, pl.ds(j * G, G)], idx_v.at[b])

  def start_idx(u, b):
    s, j = unit_su(u)
    s = jnp.minimum(s, SEQ - 1)
    pltpu.make_async_copy(
        x_hbm.at[s, pl.ds(j * G, G)], idx_v.at[b], isem[b]).start()

  def wait_idx(b):
    pltpu.make_async_copy(
        x_hbm.at[0, pl.ds(0, G)], idx_v.at[b], isem[b]).wait()

  def remap_idx(b):
    for g in range(G):
      for t in range(RG // LANES):
        iv = idx_v[b, g, pl.ds(t * LANES, LANES)]
        o = iv & 1023
        idx_v[b, g, pl.ds(t * LANES, LANES)] = (
            (iv - o) + 2 * (o & 511) + (o >> 9))

  def start_gather(b):
    for g in range(G):
      pltpu.make_async_copy(
          table_hbm.at[idx_v.at[b, g]],
          rows_v.at[b, pl.ds(g * RG, RG)],
          gsem[b]).start()

  def wait_gather(b):
    pltpu.make_async_copy(
        table_hbm.at[pl.ds(0, UR)], rows_v.at[b], gsem[b]).wait()

  iota = lax.iota(jnp.int32, LANES)
  rowvecs = [[(128 * c + 16 * li) + iota for li in range(8)]
             for c in range(G)]

  def transpose_scale(b):
    @pl.loop(0, 8)
    def _(d_hi):
      colbase = jnp.broadcast_to(8 * d_hi, (LANES,))
      for c in range(G):
        rv = rowvecs[c]
        for r in range(8):
          col = colbase + r
          vals = [plsc.load_gather(rows_v.at[b], [rv[li], col])
                  for li in range(8)]
          for li in range(8):
            phys_v[b, d_hi, c, r, pl.ds(li * LANES, LANES)] = vals[li]

  def start_store(u, b):
    s, j = unit_su(u)
    pltpu.make_async_copy(
        phys_v.at[b], out_hbm.at[s, :, pl.ds(j * G, G)], ssem[b]).start()

  def wait_store(b):
    pltpu.make_async_copy(
        phys_v.at[b], out_hbm.at[0, :, pl.ds(0, G)], ssem[b]).wait()

  sync_idx(0, 0)
  remap_idx(0)
  start_gather(0)
  sync_idx(1, 1)
  remap_idx(1)
  start_gather(1)
  wait_gather(0)
  start_idx(2, 0)
  transpose_scale(0)
  start_store(0, 0)

  @pl.loop(1, UPW - 1, step=2)
  def _(i):
    for b, off in ((1, 0), (0, 1)):
      u = i + off
      wait_store(1 - b)
      wait_idx(1 - b)
      remap_idx(1 - b)
      start_gather(1 - b)
      wait_gather(b)
      start_idx(u + 2, b)
      transpose_scale(b)
      start_store(u, b)

  wait_gather(1)
  transpose_scale(1)
  start_store(UPW - 1, 1)
  wait_idx(0)
  wait_store(0)
  wait_store(1)


def _repack_body(a_ref, o_ref):
  eye8 = SCALE * (
      lax.broadcasted_iota(jnp.int32, (MODEL_DIM, MODEL_DIM), 0)
      == lax.broadcasted_iota(jnp.int32, (MODEL_DIM, MODEL_DIM), 1)
  ).astype(jnp.float32)
  dn = (((0,), (0,)), ((), ()))
  a = a_ref[...]
  ya = lax.dot_general(a[:, 0:RCW // 2], eye8, dn,
                       preferred_element_type=jnp.float32,
                       precision=lax.Precision.HIGHEST)
  yb = lax.dot_general(a[:, RCW // 2:RCW], eye8, dn,
                       preferred_element_type=jnp.float32,
                       precision=lax.Precision.HIGHEST)
  o_ref[...] = jnp.concatenate([ya, yb], axis=1)


def _repack(table):
  table_t = jnp.swapaxes(table, 0, 1)
  nblk = PADV // RCW
  packed = pl.pallas_call(
      _repack_body,
      grid=(nblk,),
      in_specs=[pl.BlockSpec((MODEL_DIM, RCW), lambda g: (0, g))],
      out_specs=pl.BlockSpec((RCW // 2, 2 * MODEL_DIM), lambda g: (g, 0)),
      out_shape=jax.ShapeDtypeStruct((PADV // 2, 2 * MODEL_DIM),
                                     jnp.float32),
  )(table_t)
  return packed.reshape(PADV, MODEL_DIM)


@jax.jit
def _emb_lookup(x, table):
  xv = jnp.swapaxes(x, 0, 1).reshape(SEQ, BATCH // RG, RG)
  table = _repack(table)

  kern = pl.kernel(
      _emb_body,
      out_type=jax.ShapeDtypeStruct((SEQ, 8, BATCH // RG, 8, RG),
                                    jnp.float32),
      mesh=plsc.VectorSubcoreMesh(core_axis_name="c", subcore_axis_name="s"),
      compiler_params=pltpu.CompilerParams(use_tc_tiling_on_sc=False,
                                           needs_layout_passes=False),
      scratch_types=[
          pltpu.VMEM((2, G, RG), jnp.int32),
          pltpu.VMEM((2, UR, MODEL_DIM), jnp.float32),
          pltpu.VMEM((2, 8, G, 8, RG), jnp.float32),
          pltpu.SemaphoreType.DMA,
          pltpu.SemaphoreType.DMA,
          pltpu.SemaphoreType.DMA,
          pltpu.SemaphoreType.DMA,
          pltpu.SemaphoreType.DMA,
          pltpu.SemaphoreType.DMA,
      ],
  )
  out5 = kern(xv, table)
  return out5.transpose(2, 4, 0, 1, 3).reshape(BATCH, SEQ, MODEL_DIM)


def kernel(x, table):
  return _emb_lookup(x.astype(jnp.int32), table)


# speedup vs baseline: 1.4012x; 1.4012x over previous
import math

import jax
import jax.numpy as jnp
from jax import lax
from jax.experimental import pallas as pl
from jax.experimental.pallas import tpu as pltpu
from jax.experimental.pallas import tpu_sc as plsc

MODEL_DIM = 64
SCALE = math.sqrt(MODEL_DIM)

NC = 2
NS = 16
NW = NC * NS
LANES = 16

VOCAB = 1000000
RCW = 4096
RH = RCW // 2
PADV = ((VOCAB + RCW - 1) // RCW) * RCW

RG = 128
G = 2
UR = RG * G
SEQ = 200
BATCH = 4096
UNITS_PER_SEQ = BATCH // UR
N_UNITS = SEQ * UNITS_PER_SEQ
UPW = N_UNITS // NW


def _emb_body(x_hbm, table_hbm, out_hbm, idx_v, rows_v, phys_v,
              g0, g1, s0, s1, i0, i1):
  wid = lax.axis_index("s") * NC + lax.axis_index("c")
  u0 = wid * UPW

  gsem = (g0, g1)
  ssem = (s0, s1)
  isem = (i0, i1)

  def unit_su(u):
    uu = u0 + u
    return uu // UNITS_PER_SEQ, uu % UNITS_PER_SEQ

  def sync_idx(u, b):
    s, j = unit_su(u)
    pltpu.sync_copy(x_hbm.at[s, pl.ds(j * G, G)], idx_v.at[b])

  def start_idx(u, b):
    s, j = unit_su(u)
    s = jnp.minimum(s, SEQ - 1)
    pltpu.make_async_copy(
        x_hbm.at[s, pl.ds(j * G, G)], idx_v.at[b], isem[b]).start()

  def wait_idx(b):
    pltpu.make_async_copy(
        x_hbm.at[0, pl.ds(0, G)], idx_v.at[b], isem[b]).wait()

  def remap_idx(b):
    sh = RH.bit_length() - 1
    for g in range(G):
      for t in range(RG // LANES):
        iv = idx_v[b, g, pl.ds(t * LANES, LANES)]
        o = iv & (RCW - 1)
        idx_v[b, g, pl.ds(t * LANES, LANES)] = (
            (iv - o) + 2 * (o & (RH - 1)) + (o >> sh))

  def start_gather(b):
    for g in range(G):
      pltpu.make_async_copy(
          table_hbm.at[idx_v.at[b, g]],
          rows_v.at[b, pl.ds(g * RG, RG)],
          gsem[b]).start()

  def wait_gather(b):
    pltpu.make_async_copy(
        table_hbm.at[pl.ds(0, UR)], rows_v.at[b], gsem[b]).wait()

  iota = lax.iota(jnp.int32, LANES)
  rowvecs = [[(128 * c + 16 * li) + iota for li in range(8)]
             for c in range(G)]

  def transpose_scale(b):
    @pl.loop(0, 8)
    def _(d_hi):
      colbase = jnp.broadcast_to(8 * d_hi, (LANES,))
      for c in range(G):
        rv = rowvecs[c]
        for r in range(8):
          col = colbase + r
          vals = [plsc.load_gather(rows_v.at[b], [rv[li], col])
                  for li in range(8)]
          for li in range(8):
            phys_v[b, d_hi, c, r, pl.ds(li * LANES, LANES)] = vals[li]

  def start_store(u, b):
    s, j = unit_su(u)
    pltpu.make_async_copy(
        phys_v.at[b], out_hbm.at[s, :, pl.ds(j * G, G)], ssem[b]).start()

  def wait_store(b):
    pltpu.make_async_copy(
        phys_v.at[b], out_hbm.at[0, :, pl.ds(0, G)], ssem[b]).wait()

  sync_idx(0, 0)
  remap_idx(0)
  start_gather(0)
  sync_idx(1, 1)
  remap_idx(1)
  start_gather(1)
  wait_gather(0)
  start_idx(2, 0)
  transpose_scale(0)
  start_store(0, 0)

  @pl.loop(1, UPW - 1, step=2)
  def _(i):
    for b, off in ((1, 0), (0, 1)):
      u = i + off
      wait_store(1 - b)
      wait_idx(1 - b)
      remap_idx(1 - b)
      start_gather(1 - b)
      wait_gather(b)
      start_idx(u + 2, b)
      transpose_scale(b)
      start_store(u, b)

  wait_gather(1)
  transpose_scale(1)
  start_store(UPW - 1, 1)
  wait_idx(0)
  wait_store(0)
  wait_store(1)


def _repack_body(a_ref, o_ref):
  a = a_ref[...] * SCALE
  ya = a[:, 0:RCW // 2].T
  yb = a[:, RCW // 2:RCW].T
  o_ref[...] = jnp.concatenate([ya, yb], axis=1)


def _repack(table):
  table_t = jnp.swapaxes(table, 0, 1)
  nblk = PADV // RCW
  packed = pl.pallas_call(
      _repack_body,
      grid=(nblk,),
      in_specs=[pl.BlockSpec((MODEL_DIM, RCW), lambda g: (0, g))],
      out_specs=pl.BlockSpec((RCW // 2, 2 * MODEL_DIM), lambda g: (g, 0)),
      out_shape=jax.ShapeDtypeStruct((PADV // 2, 2 * MODEL_DIM),
                                     jnp.float32),
  )(table_t)
  return packed.reshape(PADV, MODEL_DIM)


@jax.jit
def _emb_lookup(x, table):
  xv = jnp.swapaxes(x, 0, 1).reshape(SEQ, BATCH // RG, RG)
  table = _repack(table)

  kern = pl.kernel(
      _emb_body,
      out_type=jax.ShapeDtypeStruct((SEQ, 8, BATCH // RG, 8, RG),
                                    jnp.float32),
      mesh=plsc.VectorSubcoreMesh(core_axis_name="c", subcore_axis_name="s"),
      compiler_params=pltpu.CompilerParams(use_tc_tiling_on_sc=False,
                                           needs_layout_passes=False),
      scratch_types=[
          pltpu.VMEM((2, G, RG), jnp.int32),
          pltpu.VMEM((2, UR, MODEL_DIM), jnp.float32),
          pltpu.VMEM((2, 8, G, 8, RG), jnp.float32),
          pltpu.SemaphoreType.DMA,
          pltpu.SemaphoreType.DMA,
          pltpu.SemaphoreType.DMA,
          pltpu.SemaphoreType.DMA,
          pltpu.SemaphoreType.DMA,
          pltpu.SemaphoreType.DMA,
      ],
  )
  out5 = kern(xv, table)
  return out5.transpose(2, 4, 0, 1, 3).reshape(BATCH, SEQ, MODEL_DIM)


def kernel(x, table):
  return _emb_lookup(x.astype(jnp.int32), table)


# speedup vs baseline: 3.0140x; 2.1510x over previous
import math

import jax
import jax.numpy as jnp
from jax import lax
from jax.experimental import pallas as pl
from jax.experimental.pallas import tpu as pltpu
from jax.experimental.pallas import tpu_sc as plsc

MODEL_DIM = 64
SCALE = math.sqrt(MODEL_DIM)

NC = 2
NS = 16
NW = NC * NS
LANES = 16

VOCAB = 1000000
RCW = 4096
RH = RCW // 2
PADV = ((VOCAB + RCW - 1) // RCW) * RCW

RG = 128
G = 2
UR = RG * G
SEQ = 200
BATCH = 4096
UNITS_PER_SEQ = BATCH // UR
N_UNITS = SEQ * UNITS_PER_SEQ
UPW = N_UNITS // NW


def _emb_body(x_hbm, table_hbm, out_hbm, idx_v, rows_v, phys_v,
              g0, g1, s0, s1, i0, i1):
  wid = lax.axis_index("s") * NC + lax.axis_index("c")
  u0 = wid * UPW

  gsem = (g0, g1)
  ssem = (s0, s1)
  isem = (i0, i1)

  def unit_su(u):
    uu = u0 + u
    return uu // UNITS_PER_SEQ, uu % UNITS_PER_SEQ

  def sync_idx(u, b):
    s, j = unit_su(u)
    pltpu.sync_copy(x_hbm.at[s, pl.ds(j * G, G)], idx_v.at[b])

  def start_idx(u, b):
    s, j = unit_su(u)
    s = jnp.minimum(s, SEQ - 1)
    pltpu.make_async_copy(
        x_hbm.at[s, pl.ds(j * G, G)], idx_v.at[b], isem[b]).start()

  def wait_idx(b):
    pltpu.make_async_copy(
        x_hbm.at[0, pl.ds(0, G)], idx_v.at[b], isem[b]).wait()

  def remap_idx(b):
    sh = RH.bit_length() - 1
    for g in range(G):
      for t in range(RG // LANES):
        iv = idx_v[b, g, pl.ds(t * LANES, LANES)]
        o = iv & (RCW - 1)
        idx_v[b, g, pl.ds(t * LANES, LANES)] = (
            (iv - o) + 2 * (o & (RH - 1)) + (o >> sh))

  def start_gather(b):
    for g in range(G):
      pltpu.make_async_copy(
          table_hbm.at[idx_v.at[b, g]],
          rows_v.at[b, pl.ds(g * RG, RG)],
          gsem[b]).start()

  def wait_gather(b):
    pltpu.make_async_copy(
        table_hbm.at[pl.ds(0, UR)], rows_v.at[b], gsem[b]).wait()

  iota = lax.iota(jnp.int32, LANES)
  zkv = [(iota + k) & 15 for k in range(LANES)]

  def transpose_scale(b):
    @pl.loop(0, 32)
    def _(q):
      lb16 = (q >> 2) << 4
      fb = q & 3
      f0 = jnp.broadcast_to(fb << 4, (LANES,))
      dd0 = jnp.broadcast_to(fb << 1, (LANES,))
      for c in range(G):
        rowv = iota + (128 * c + lb16)
        lidx = iota + lb16
        cidx = jnp.broadcast_to(c, (LANES,))
        vals = []
        for k in range(LANES):
          col = zkv[k] + f0
          vals.append(plsc.load_gather(rows_v.at[b], [rowv, col]))
        for k in range(LANES):
          t = zkv[k] >> 3
          didx = t + dd0
          ridx = zkv[k] - (t << 3)
          plsc.store_scatter(phys_v.at[b], [didx, cidx, ridx, lidx],
                             vals[k])

  def start_store(u, b):
    s, j = unit_su(u)
    pltpu.make_async_copy(
        phys_v.at[b], out_hbm.at[s, :, pl.ds(j * G, G)], ssem[b]).start()

  def wait_store(b):
    pltpu.make_async_copy(
        phys_v.at[b], out_hbm.at[0, :, pl.ds(0, G)], ssem[b]).wait()

  sync_idx(0, 0)
  remap_idx(0)
  start_gather(0)
  sync_idx(1, 1)
  remap_idx(1)
  start_gather(1)
  wait_gather(0)
  start_idx(2, 0)
  transpose_scale(0)
  start_store(0, 0)

  @pl.loop(1, UPW - 1, step=2)
  def _(i):
    for b, off in ((1, 0), (0, 1)):
      u = i + off
      wait_store(1 - b)
      wait_idx(1 - b)
      remap_idx(1 - b)
      start_gather(1 - b)
      wait_gather(b)
      start_idx(u + 2, b)
      transpose_scale(b)
      start_store(u, b)

  wait_gather(1)
  transpose_scale(1)
  start_store(UPW - 1, 1)
  wait_idx(0)
  wait_store(0)
  wait_store(1)


def _repack_body(a_ref, o_ref):
  a = a_ref[...] * SCALE
  ya = a[:, 0:RCW // 2].T
  yb = a[:, RCW // 2:RCW].T
  o_ref[...] = jnp.concatenate([ya, yb], axis=1)


def _repack(table):
  table_t = jnp.swapaxes(table, 0, 1)
  nblk = PADV // RCW
  packed = pl.pallas_call(
      _repack_body,
      grid=(nblk,),
      in_specs=[pl.BlockSpec((MODEL_DIM, RCW), lambda g: (0, g))],
      out_specs=pl.BlockSpec((RCW // 2, 2 * MODEL_DIM), lambda g: (g, 0)),
      out_shape=jax.ShapeDtypeStruct((PADV // 2, 2 * MODEL_DIM),
                                     jnp.float32),
  )(table_t)
  return packed.reshape(PADV, MODEL_DIM)


@jax.jit
def _emb_lookup(x, table):
  xv = jnp.swapaxes(x, 0, 1).reshape(SEQ, BATCH // RG, RG)
  table = _repack(table)

  kern = pl.kernel(
      _emb_body,
      out_type=jax.ShapeDtypeStruct((SEQ, 8, BATCH // RG, 8, RG),
                                    jnp.float32),
      mesh=plsc.VectorSubcoreMesh(core_axis_name="c", subcore_axis_name="s"),
      compiler_params=pltpu.CompilerParams(use_tc_tiling_on_sc=False,
                                           needs_layout_passes=False),
      scratch_types=[
          pltpu.VMEM((2, G, RG), jnp.int32),
          pltpu.VMEM((2, UR, MODEL_DIM), jnp.float32),
          pltpu.VMEM((2, 8, G, 8, RG), jnp.float32),
          pltpu.SemaphoreType.DMA,
          pltpu.SemaphoreType.DMA,
          pltpu.SemaphoreType.DMA,
          pltpu.SemaphoreType.DMA,
          pltpu.SemaphoreType.DMA,
          pltpu.SemaphoreType.DMA,
      ],
  )
  out5 = kern(xv, table)
  return out5.transpose(2, 4, 0, 1, 3).reshape(BATCH, SEQ, MODEL_DIM)


def kernel(x, table):
  return _emb_lookup(x.astype(jnp.int32), table)


# speedup vs baseline: 3.3562x; 1.1135x over previous
import math

import jax
import jax.numpy as jnp
from jax import lax
from jax.experimental import pallas as pl
from jax.experimental.pallas import tpu as pltpu
from jax.experimental.pallas import tpu_sc as plsc

MODEL_DIM = 64
SCALE = math.sqrt(MODEL_DIM)

NC = 2
NS = 16
NW = NC * NS
LANES = 16

VOCAB = 1000000
RCW = 8192
RH = RCW // 2
PADV = ((VOCAB + RCW - 1) // RCW) * RCW

RG = 128
G = 2
UR = RG * G
SEQ = 200
BATCH = 4096
UNITS_PER_SEQ = BATCH // UR
N_UNITS = SEQ * UNITS_PER_SEQ
UPW = N_UNITS // NW


def _emb_body(x_hbm, table_hbm, out_hbm, idx_v, rows_v, phys_v,
              g0, g1, s0, s1, i0, i1):
  wid = lax.axis_index("s") * NC + lax.axis_index("c")
  u0 = wid * UPW

  gsem = (g0, g1)
  ssem = (s0, s1)
  isem = (i0, i1)

  def unit_su(u):
    uu = u0 + u
    return uu // UNITS_PER_SEQ, uu % UNITS_PER_SEQ

  def sync_idx(u, b):
    s, j = unit_su(u)
    pltpu.sync_copy(x_hbm.at[s, pl.ds(j * G, G)], idx_v.at[b])

  def start_idx(u, b):
    s, j = unit_su(u)
    s = jnp.minimum(s, SEQ - 1)
    pltpu.make_async_copy(
        x_hbm.at[s, pl.ds(j * G, G)], idx_v.at[b], isem[b]).start()

  def wait_idx(b):
    pltpu.make_async_copy(
        x_hbm.at[0, pl.ds(0, G)], idx_v.at[b], isem[b]).wait()

  def remap_idx(b):
    sh = RH.bit_length() - 1
    for g in range(G):
      for t in range(RG // LANES):
        iv = idx_v[b, g, pl.ds(t * LANES, LANES)]
        o = iv & (RCW - 1)
        idx_v[b, g, pl.ds(t * LANES, LANES)] = (
            (iv - o) + 2 * (o & (RH - 1)) + (o >> sh))

  def start_gather(b):
    for g in range(G):
      pltpu.make_async_copy(
          table_hbm.at[idx_v.at[b, g]],
          rows_v.at[b, pl.ds(g * RG, RG)],
          gsem[b]).start()

  def wait_gather(b):
    pltpu.make_async_copy(
        table_hbm.at[pl.ds(0, UR)], rows_v.at[b], gsem[b]).wait()

  iota = lax.iota(jnp.int32, LANES)
  zkv = [(iota + k) & 15 for k in range(LANES)]

  def transpose_scale(b):
    @pl.loop(0, 32)
    def _(q):
      lb16 = (q >> 2) << 4
      fb = q & 3
      f0 = jnp.broadcast_to(fb << 4, (LANES,))
      dd0 = jnp.broadcast_to(fb << 1, (LANES,))
      for c in range(G):
        rowv = iota + (128 * c + lb16)
        lidx = iota + lb16
        cidx = jnp.broadcast_to(c, (LANES,))
        vals = []
        for k in range(LANES):
          col = zkv[k] + f0
          vals.append(plsc.load_gather(rows_v.at[b], [rowv, col]))
        for k in range(LANES):
          t = zkv[k] >> 3
          didx = t + dd0
          ridx = zkv[k] - (t << 3)
          plsc.store_scatter(phys_v.at[b], [didx, cidx, ridx, lidx],
                             vals[k])

  def start_store(u, b):
    s, j = unit_su(u)
    pltpu.make_async_copy(
        phys_v.at[b], out_hbm.at[s, :, pl.ds(j * G, G)], ssem[b]).start()

  def wait_store(b):
    pltpu.make_async_copy(
        phys_v.at[b], out_hbm.at[0, :, pl.ds(0, G)], ssem[b]).wait()

  sync_idx(0, 0)
  remap_idx(0)
  start_gather(0)
  sync_idx(1, 1)
  remap_idx(1)
  start_gather(1)
  wait_gather(0)
  start_idx(2, 0)
  transpose_scale(0)
  start_store(0, 0)

  @pl.loop(1, UPW - 1, step=2)
  def _(i):
    for b, off in ((1, 0), (0, 1)):
      u = i + off
      wait_store(1 - b)
      wait_idx(1 - b)
      remap_idx(1 - b)
      start_gather(1 - b)
      wait_gather(b)
      start_idx(u + 2, b)
      transpose_scale(b)
      start_store(u, b)

  wait_gather(1)
  transpose_scale(1)
  start_store(UPW - 1, 1)
  wait_idx(0)
  wait_store(0)
  wait_store(1)


def _repack_body(a_ref, o_ref):
  a = a_ref[...] * SCALE
  ya = a[:, 0:RCW // 2].T
  yb = a[:, RCW // 2:RCW].T
  o_ref[...] = jnp.concatenate([ya, yb], axis=1)


def _repack(table):
  table_t = jnp.swapaxes(table, 0, 1)
  nblk = PADV // RCW
  packed = pl.pallas_call(
      _repack_body,
      grid=(nblk,),
      in_specs=[pl.BlockSpec((MODEL_DIM, RCW), lambda g: (0, g))],
      out_specs=pl.BlockSpec((RCW // 2, 2 * MODEL_DIM), lambda g: (g, 0)),
      out_shape=jax.ShapeDtypeStruct((PADV // 2, 2 * MODEL_DIM),
                                     jnp.float32),
  )(table_t)
  return packed.reshape(PADV, MODEL_DIM)


@jax.jit
def _emb_lookup(x, table):
  xv = jnp.swapaxes(x, 0, 1).reshape(SEQ, BATCH // RG, RG)
  table = _repack(table)

  kern = pl.kernel(
      _emb_body,
      out_type=jax.ShapeDtypeStruct((SEQ, 8, BATCH // RG, 8, RG),
                                    jnp.float32),
      mesh=plsc.VectorSubcoreMesh(core_axis_name="c", subcore_axis_name="s"),
      compiler_params=pltpu.CompilerParams(use_tc_tiling_on_sc=False,
                                           needs_layout_passes=False),
      scratch_types=[
          pltpu.VMEM((2, G, RG), jnp.int32),
          pltpu.VMEM((2, UR, MODEL_DIM), jnp.float32),
          pltpu.VMEM((2, 8, G, 8, RG), jnp.float32),
          pltpu.SemaphoreType.DMA,
          pltpu.SemaphoreType.DMA,
          pltpu.SemaphoreType.DMA,
          pltpu.SemaphoreType.DMA,
          pltpu.SemaphoreType.DMA,
          pltpu.SemaphoreType.DMA,
      ],
  )
  out5 = kern(xv, table)
  return out5.transpose(2, 4, 0, 1, 3).reshape(BATCH, SEQ, MODEL_DIM)


def kernel(x, table):
  return _emb_lookup(x.astype(jnp.int32), table)


# speedup vs baseline: 3.5677x; 1.0630x over previous
import math

import jax
import jax.numpy as jnp
from jax import lax
from jax.experimental import pallas as pl
from jax.experimental.pallas import tpu as pltpu
from jax.experimental.pallas import tpu_sc as plsc

MODEL_DIM = 64
SCALE = math.sqrt(MODEL_DIM)

NC = 2
NS = 16
NW = NC * NS
LANES = 16

VOCAB = 1000000
RCW = 16384
RH = RCW // 2
PADV = ((VOCAB + RCW - 1) // RCW) * RCW

RG = 128
G = 2
UR = RG * G
SEQ = 200
BATCH = 4096
UNITS_PER_SEQ = BATCH // UR
N_UNITS = SEQ * UNITS_PER_SEQ
UPW = N_UNITS // NW


def _emb_body(x_hbm, table_hbm, out_hbm, idx_v, rows_v, phys_v,
              g0, g1, s0, s1, i0, i1):
  wid = lax.axis_index("s") * NC + lax.axis_index("c")
  u0 = wid * UPW

  gsem = (g0, g1)
  ssem = (s0, s1)
  isem = (i0, i1)

  def unit_su(u):
    uu = u0 + u
    return uu // UNITS_PER_SEQ, uu % UNITS_PER_SEQ

  def sync_idx(u, b):
    s, j = unit_su(u)
    pltpu.sync_copy(x_hbm.at[s, pl.ds(j * G, G)], idx_v.at[b])

  def start_idx(u, b):
    s, j = unit_su(u)
    s = jnp.minimum(s, SEQ - 1)
    pltpu.make_async_copy(
        x_hbm.at[s, pl.ds(j * G, G)], idx_v.at[b], isem[b]).start()

  def wait_idx(b):
    pltpu.make_async_copy(
        x_hbm.at[0, pl.ds(0, G)], idx_v.at[b], isem[b]).wait()

  def remap_idx(b):
    sh = RH.bit_length() - 1
    for g in range(G):
      for t in range(RG // LANES):
        iv = idx_v[b, g, pl.ds(t * LANES, LANES)]
        o = iv & (RCW - 1)
        idx_v[b, g, pl.ds(t * LANES, LANES)] = (
            (iv - o) + 2 * (o & (RH - 1)) + (o >> sh))

  def start_gather(b):
    for g in range(G):
      pltpu.make_async_copy(
          table_hbm.at[idx_v.at[b, g]],
          rows_v.at[b, pl.ds(g * RG, RG)],
          gsem[b]).start()

  def wait_gather(b):
    pltpu.make_async_copy(
        table_hbm.at[pl.ds(0, UR)], rows_v.at[b], gsem[b]).wait()

  iota = lax.iota(jnp.int32, LANES)
  zkv = [(iota + k) & 15 for k in range(LANES)]

  def transpose_scale(b):
    @pl.loop(0, 32)
    def _(q):
      lb16 = (q >> 2) << 4
      fb = q & 3
      f0 = jnp.broadcast_to(fb << 4, (LANES,))
      dd0 = jnp.broadcast_to(fb << 1, (LANES,))
      for c in range(G):
        rowv = iota + (128 * c + lb16)
        lidx = iota + lb16
        cidx = jnp.broadcast_to(c, (LANES,))
        vals = []
        for k in range(LANES):
          col = zkv[k] + f0
          vals.append(plsc.load_gather(rows_v.at[b], [rowv, col]))
        for k in range(LANES):
          t = zkv[k] >> 3
          didx = t + dd0
          ridx = zkv[k] - (t << 3)
          plsc.store_scatter(phys_v.at[b], [didx, cidx, ridx, lidx],
                             vals[k])

  def start_store(u, b):
    s, j = unit_su(u)
    pltpu.make_async_copy(
        phys_v.at[b], out_hbm.at[s, :, pl.ds(j * G, G)], ssem[b]).start()

  def wait_store(b):
    pltpu.make_async_copy(
        phys_v.at[b], out_hbm.at[0, :, pl.ds(0, G)], ssem[b]).wait()

  sync_idx(0, 0)
  remap_idx(0)
  start_gather(0)
  sync_idx(1, 1)
  remap_idx(1)
  start_gather(1)
  wait_gather(0)
  start_idx(2, 0)
  transpose_scale(0)
  start_store(0, 0)

  @pl.loop(1, UPW - 1, step=2)
  def _(i):
    for b, off in ((1, 0), (0, 1)):
      u = i + off
      wait_store(1 - b)
      wait_idx(1 - b)
      remap_idx(1 - b)
      start_gather(1 - b)
      wait_gather(b)
      start_idx(u + 2, b)
      transpose_scale(b)
      start_store(u, b)

  wait_gather(1)
  transpose_scale(1)
  start_store(UPW - 1, 1)
  wait_idx(0)
  wait_store(0)
  wait_store(1)


def _repack_body(a_ref, o_ref):
  a = a_ref[...] * SCALE
  ya = a[:, 0:RCW // 2].T
  yb = a[:, RCW // 2:RCW].T
  o_ref[...] = jnp.concatenate([ya, yb], axis=1)


def _repack(table):
  table_t = jnp.swapaxes(table, 0, 1)
  nblk = PADV // RCW
  packed = pl.pallas_call(
      _repack_body,
      grid=(nblk,),
      in_specs=[pl.BlockSpec((MODEL_DIM, RCW), lambda g: (0, g))],
      out_specs=pl.BlockSpec((RCW // 2, 2 * MODEL_DIM), lambda g: (g, 0)),
      out_shape=jax.ShapeDtypeStruct((PADV // 2, 2 * MODEL_DIM),
                                     jnp.float32),
  )(table_t)
  return packed.reshape(PADV, MODEL_DIM)


@jax.jit
def _emb_lookup(x, table):
  xv = jnp.swapaxes(x, 0, 1).reshape(SEQ, BATCH // RG, RG)
  table = _repack(table)

  kern = pl.kernel(
      _emb_body,
      out_type=jax.ShapeDtypeStruct((SEQ, 8, BATCH // RG, 8, RG),
                                    jnp.float32),
      mesh=plsc.VectorSubcoreMesh(core_axis_name="c", subcore_axis_name="s"),
      compiler_params=pltpu.CompilerParams(use_tc_tiling_on_sc=False,
                                           needs_layout_passes=False),
      scratch_types=[
          pltpu.VMEM((2, G, RG), jnp.int32),
          pltpu.VMEM((2, UR, MODEL_DIM), jnp.float32),
          pltpu.VMEM((2, 8, G, 8, RG), jnp.float32),
          pltpu.SemaphoreType.DMA,
          pltpu.SemaphoreType.DMA,
          pltpu.SemaphoreType.DMA,
          pltpu.SemaphoreType.DMA,
          pltpu.SemaphoreType.DMA,
          pltpu.SemaphoreType.DMA,
      ],
  )
  out5 = kern(xv, table)
  return out5.transpose(2, 4, 0, 1, 3).reshape(BATCH, SEQ, MODEL_DIM)


def kernel(x, table):
  return _emb_lookup(x.astype(jnp.int32), table)


# speedup vs baseline: 3.6565x; 1.0249x over previous
import math

import jax
import jax.numpy as jnp
from jax import lax
from jax.experimental import pallas as pl
from jax.experimental.pallas import tpu as pltpu
from jax.experimental.pallas import tpu_sc as plsc

MODEL_DIM = 64
SCALE = math.sqrt(MODEL_DIM)

NC = 2
NS = 16
NW = NC * NS
LANES = 16

VOCAB = 1000000
RCW = 32768
RH = RCW // 2
PADV = ((VOCAB + RCW - 1) // RCW) * RCW

RG = 128
G = 2
UR = RG * G
SEQ = 200
BATCH = 4096
UNITS_PER_SEQ = BATCH // UR
N_UNITS = SEQ * UNITS_PER_SEQ
UPW = N_UNITS // NW


def _emb_body(x_hbm, table_hbm, out_hbm, idx_v, rows_v, phys_v,
              g0, g1, s0, s1, i0, i1):
  wid = lax.axis_index("s") * NC + lax.axis_index("c")
  u0 = wid * UPW

  gsem = (g0, g1)
  ssem = (s0, s1)
  isem = (i0, i1)

  def unit_su(u):
    uu = u0 + u
    return uu // UNITS_PER_SEQ, uu % UNITS_PER_SEQ

  def sync_idx(u, b):
    s, j = unit_su(u)
    pltpu.sync_copy(x_hbm.at[s, pl.ds(j * G, G)], idx_v.at[b])

  def start_idx(u, b):
    s, j = unit_su(u)
    s = jnp.minimum(s, SEQ - 1)
    pltpu.make_async_copy(
        x_hbm.at[s, pl.ds(j * G, G)], idx_v.at[b], isem[b]).start()

  def wait_idx(b):
    pltpu.make_async_copy(
        x_hbm.at[0, pl.ds(0, G)], idx_v.at[b], isem[b]).wait()

  def remap_idx(b):
    sh = RH.bit_length() - 1
    for g in range(G):
      for t in range(RG // LANES):
        iv = idx_v[b, g, pl.ds(t * LANES, LANES)]
        o = iv & (RCW - 1)
        idx_v[b, g, pl.ds(t * LANES, LANES)] = (
            (iv - o) + 2 * (o & (RH - 1)) + (o >> sh))

  def start_gather(b):
    for g in range(G):
      pltpu.make_async_copy(
          table_hbm.at[idx_v.at[b, g]],
          rows_v.at[b, pl.ds(g * RG, RG)],
          gsem[b]).start()

  def wait_gather(b):
    pltpu.make_async_copy(
        table_hbm.at[pl.ds(0, UR)], rows_v.at[b], gsem[b]).wait()

  iota = lax.iota(jnp.int32, LANES)
  zkv = [(iota + k) & 15 for k in range(LANES)]

  def transpose_scale(b):
    @pl.loop(0, 32)
    def _(q):
      lb16 = (q >> 2) << 4
      fb = q & 3
      f0 = jnp.broadcast_to(fb << 4, (LANES,))
      dd0 = jnp.broadcast_to(fb << 1, (LANES,))
      for c in range(G):
        rowv = iota + (128 * c + lb16)
        lidx = iota + lb16
        cidx = jnp.broadcast_to(c, (LANES,))
        vals = []
        for k in range(LANES):
          col = zkv[k] + f0
          vals.append(plsc.load_gather(rows_v.at[b], [rowv, col]))
        for k in range(LANES):
          t = zkv[k] >> 3
          didx = t + dd0
          ridx = zkv[k] - (t << 3)
          plsc.store_scatter(phys_v.at[b], [didx, cidx, ridx, lidx],
                             vals[k])

  def start_store(u, b):
    s, j = unit_su(u)
    pltpu.make_async_copy(
        phys_v.at[b], out_hbm.at[s, :, pl.ds(j * G, G)], ssem[b]).start()

  def wait_store(b):
    pltpu.make_async_copy(
        phys_v.at[b], out_hbm.at[0, :, pl.ds(0, G)], ssem[b]).wait()

  sync_idx(0, 0)
  remap_idx(0)
  start_gather(0)
  sync_idx(1, 1)
  remap_idx(1)
  start_gather(1)
  wait_gather(0)
  start_idx(2, 0)
  transpose_scale(0)
  start_store(0, 0)

  @pl.loop(1, UPW - 1, step=2)
  def _(i):
    for b, off in ((1, 0), (0, 1)):
      u = i + off
      wait_store(1 - b)
      wait_idx(1 - b)
      remap_idx(1 - b)
      start_gather(1 - b)
      wait_gather(b)
      start_idx(u + 2, b)
      transpose_scale(b)
      start_store(u, b)

  wait_gather(1)
  transpose_scale(1)
  start_store(UPW - 1, 1)
  wait_idx(0)
  wait_store(0)
  wait_store(1)


def _repack_body(a_ref, o_ref):
  a = a_ref[...] * SCALE
  ya = a[:, 0:RCW // 2].T
  yb = a[:, RCW // 2:RCW].T
  o_ref[...] = jnp.concatenate([ya, yb], axis=1)


def _repack(table):
  table_t = jnp.swapaxes(table, 0, 1)
  nblk = PADV // RCW
  packed = pl.pallas_call(
      _repack_body,
      grid=(nblk,),
      in_specs=[pl.BlockSpec((MODEL_DIM, RCW), lambda g: (0, g))],
      out_specs=pl.BlockSpec((RCW // 2, 2 * MODEL_DIM), lambda g: (g, 0)),
      out_shape=jax.ShapeDtypeStruct((PADV // 2, 2 * MODEL_DIM),
                                     jnp.float32),
  )(table_t)
  return packed.reshape(PADV, MODEL_DIM)


@jax.jit
def _emb_lookup(x, table):
  xv = jnp.swapaxes(x, 0, 1).reshape(SEQ, BATCH // RG, RG)
  table = _repack(table)

  kern = pl.kernel(
      _emb_body,
      out_type=jax.ShapeDtypeStruct((SEQ, 8, BATCH // RG, 8, RG),
                                    jnp.float32),
      mesh=plsc.VectorSubcoreMesh(core_axis_name="c", subcore_axis_name="s"),
      compiler_params=pltpu.CompilerParams(use_tc_tiling_on_sc=False,
                                           needs_layout_passes=False),
      scratch_types=[
          pltpu.VMEM((2, G, RG), jnp.int32),
          pltpu.VMEM((2, UR, MODEL_DIM), jnp.float32),
          pltpu.VMEM((2, 8, G, 8, RG), jnp.float32),
          pltpu.SemaphoreType.DMA,
          pltpu.SemaphoreType.DMA,
          pltpu.SemaphoreType.DMA,
          pltpu.SemaphoreType.DMA,
          pltpu.SemaphoreType.DMA,
          pltpu.SemaphoreType.DMA,
      ],
  )
  out5 = kern(xv, table)
  return out5.transpose(2, 4, 0, 1, 3).reshape(BATCH, SEQ, MODEL_DIM)


def kernel(x, table):
  return _emb_lookup(x.astype(jnp.int32), table)
